# baseline (speedup 1.0000x reference)
import jax
import jax.numpy as jnp
from jax.experimental import pallas as pl
from jax.experimental.pallas import tpu as pltpu

_BM = 20000


def _heads_kernel(x_ref, w_ref, b_ref, cls_ref, box_ref, obj_ref):
    acc = jnp.dot(x_ref[...], w_ref[...], preferred_element_type=jnp.float32)
    acc = acc + b_ref[...]
    n_cls = cls_ref.shape[1]
    n_box = box_ref.shape[1]
    cls_ref[...] = acc[:, :n_cls]
    box_ref[...] = acc[:, n_cls:n_cls + n_box]
    obj_ref[...] = acc[:, n_cls + n_box:n_cls + n_box + obj_ref.shape[1]]


def kernel(x, W_cls, b_cls, W_box, b_box, W_obj, b_obj):
    M, K = x.shape
    n_cls = W_cls.shape[1]
    n_box = W_box.shape[1]
    n_obj = W_obj.shape[1]
    n_all = n_cls + n_box + n_obj

    W = jnp.concatenate([W_cls, W_box, W_obj], axis=1)
    b = jnp.concatenate([b_cls, b_box, b_obj])[None, :]

    bm = _BM if M % _BM == 0 else M
    grid = (M // bm,)

    cls_out, box_out, obj_out = pl.pallas_call(
        _heads_kernel,
        grid=grid,
        in_specs=[
            pl.BlockSpec((bm, K), lambda i: (i, 0)),
            pl.BlockSpec((K, n_all), lambda i: (0, 0)),
            pl.BlockSpec((1, n_all), lambda i: (0, 0)),
        ],
        out_specs=[
            pl.BlockSpec((bm, n_cls), lambda i: (i, 0)),
            pl.BlockSpec((bm, n_box), lambda i: (i, 0)),
            pl.BlockSpec((bm, n_obj), lambda i: (i, 0)),
        ],
        out_shape=[
            jax.ShapeDtypeStruct((M, n_cls), x.dtype),
            jax.ShapeDtypeStruct((M, n_box), x.dtype),
            jax.ShapeDtypeStruct((M, n_obj), x.dtype),
        ],
        compiler_params=pltpu.CompilerParams(
            dimension_semantics=("parallel",),
            vmem_limit_bytes=67108864,
        ),
    )(x, W, b)
    return (cls_out, box_out, obj_out)


# speedup vs baseline: 1.0748x; 1.0748x over previous
import jax
import jax.numpy as jnp
from jax.experimental import pallas as pl
from jax.experimental.pallas import tpu as pltpu

_NC = 20


def _heads_kernel(x_hbm, w_ref, b_ref, cls_hbm, box_hbm, obj_hbm,
                  x_v, cls_v, box_v, obj_v, in_sem, out_sem):
    n_rows = x_hbm.shape[0]
    nc = _NC
    rows = n_rows // nc
    n_cls = cls_hbm.shape[1]
    n_box = box_hbm.shape[1]

    def in_copy(c):
        sl = pl.ds(c * rows, rows)
        return pltpu.make_async_copy(x_hbm.at[sl, :], x_v.at[sl, :], in_sem.at[c])

    def out_copies(c):
        sl = pl.ds(c * rows, rows)
        return (
            pltpu.make_async_copy(cls_v.at[sl, :], cls_hbm.at[sl, :], out_sem.at[c, 0]),
            pltpu.make_async_copy(box_v.at[sl, :], box_hbm.at[sl, :], out_sem.at[c, 1]),
            pltpu.make_async_copy(obj_v.at[sl, :], obj_hbm.at[sl, :], out_sem.at[c, 2]),
        )

    for c in range(nc):
        in_copy(c).start()

    w = w_ref[...]
    b = b_ref[...]
    for c in range(nc):
        in_copy(c).wait()
        sl = pl.ds(c * rows, rows)
        acc = jnp.dot(x_v[sl, :], w, preferred_element_type=jnp.float32) + b
        cls_v[sl, :] = acc[:, :n_cls]
        box_v[sl, :] = acc[:, n_cls:n_cls + n_box]
        obj_v[sl, :] = acc[:, n_cls + n_box:]
        for cp in out_copies(c):
            cp.start()

    for c in range(nc):
        for cp in out_copies(c):
            cp.wait()


def kernel(x, W_cls, b_cls, W_box, b_box, W_obj, b_obj):
    M, K = x.shape
    n_cls = W_cls.shape[1]
    n_box = W_box.shape[1]
    n_obj = W_obj.shape[1]
    n_all = n_cls + n_box + n_obj

    W = jnp.concatenate([W_cls, W_box, W_obj], axis=1)
    b = jnp.concatenate([b_cls, b_box, b_obj])[None, :]

    cls_out, box_out, obj_out = pl.pallas_call(
        _heads_kernel,
        in_specs=[
            pl.BlockSpec(memory_space=pltpu.HBM),
            pl.BlockSpec(memory_space=pltpu.VMEM),
            pl.BlockSpec(memory_space=pltpu.VMEM),
        ],
        out_specs=[
            pl.BlockSpec(memory_space=pltpu.HBM),
            pl.BlockSpec(memory_space=pltpu.HBM),
            pl.BlockSpec(memory_space=pltpu.HBM),
        ],
        out_shape=[
            jax.ShapeDtypeStruct((M, n_cls), x.dtype),
            jax.ShapeDtypeStruct((M, n_box), x.dtype),
            jax.ShapeDtypeStruct((M, n_obj), x.dtype),
        ],
        scratch_shapes=[
            pltpu.VMEM((M, K), jnp.float32),
            pltpu.VMEM((M, n_cls), jnp.float32),
            pltpu.VMEM((M, n_box), jnp.float32),
            pltpu.VMEM((M, n_obj), jnp.float32),
            pltpu.SemaphoreType.DMA((_NC,)),
            pltpu.SemaphoreType.DMA((_NC, 3)),
        ],
        compiler_params=pltpu.CompilerParams(
            vmem_limit_bytes=67108864,
        ),
    )(x, W, b)
    return (cls_out, box_out, obj_out)


# speedup vs baseline: 1.0757x; 1.0008x over previous
import jax
import jax.numpy as jnp
from jax.experimental import pallas as pl
from jax.experimental.pallas import tpu as pltpu

_NC = 20
_NXB = 4
_NOB = 2


def _heads_kernel(x_hbm, w_ref, b_ref, cls_hbm, box_hbm, obj_hbm,
                  x_v, cls_v, box_v, obj_v, in_sem, out_sem):
    n_rows = x_hbm.shape[0]
    nc = _NC
    rows = n_rows // nc
    n_cls = cls_hbm.shape[1]
    n_box = box_hbm.shape[1]

    def in_copy(c):
        jb, off = c % _NXB, (c // _NXB) * rows
        return pltpu.make_async_copy(
            x_hbm.at[pl.ds(c * rows, rows), :],
            x_v.at[jb, pl.ds(off, rows), :],
            in_sem.at[c])

    def out_copies(c):
        jb, off = c % _NOB, (c // _NOB) * rows
        sl = pl.ds(c * rows, rows)
        vs = pl.ds(off, rows)
        return (
            pltpu.make_async_copy(cls_v.at[jb, vs, :], cls_hbm.at[sl, :], out_sem.at[c, 0]),
            pltpu.make_async_copy(box_v.at[jb, vs, :], box_hbm.at[sl, :], out_sem.at[c, 1]),
            pltpu.make_async_copy(obj_v.at[jb, vs, :], obj_hbm.at[sl, :], out_sem.at[c, 2]),
        )

    for c in range(nc):
        in_copy(c).start()

    w = w_ref[...]
    b = b_ref[...]
    for c in range(nc):
        in_copy(c).wait()
        jb, off = c % _NXB, (c // _NXB) * rows
        ob, ooff = c % _NOB, (c // _NOB) * rows
        acc = jnp.dot(x_v[jb, pl.ds(off, rows), :], w,
                      preferred_element_type=jnp.float32) + b
        vs = pl.ds(ooff, rows)
        cls_v[ob, vs, :] = acc[:, :n_cls]
        box_v[ob, vs, :] = acc[:, n_cls:n_cls + n_box]
        obj_v[ob, vs, :] = acc[:, n_cls + n_box:]
        for cp in out_copies(c):
            cp.start()

    for c in range(nc):
        for cp in out_copies(c):
            cp.wait()


def kernel(x, W_cls, b_cls, W_box, b_box, W_obj, b_obj):
    M, K = x.shape
    n_cls = W_cls.shape[1]
    n_box = W_box.shape[1]
    n_obj = W_obj.shape[1]

    W = jnp.concatenate([W_cls, W_box, W_obj], axis=1)
    b = jnp.concatenate([b_cls, b_box, b_obj])[None, :]

    cls_out, box_out, obj_out = pl.pallas_call(
        _heads_kernel,
        in_specs=[
            pl.BlockSpec(memory_space=pltpu.HBM),
            pl.BlockSpec(memory_space=pltpu.VMEM),
            pl.BlockSpec(memory_space=pltpu.VMEM),
        ],
        out_specs=[
            pl.BlockSpec(memory_space=pltpu.HBM),
            pl.BlockSpec(memory_space=pltpu.HBM),
            pl.BlockSpec(memory_space=pltpu.HBM),
        ],
        out_shape=[
            jax.ShapeDtypeStruct((M, n_cls), x.dtype),
            jax.ShapeDtypeStruct((M, n_box), x.dtype),
            jax.ShapeDtypeStruct((M, n_obj), x.dtype),
        ],
        scratch_shapes=[
            pltpu.VMEM((_NXB, M // _NXB, K), jnp.float32),
            pltpu.VMEM((_NOB, M // _NOB, n_cls), jnp.float32),
            pltpu.VMEM((_NOB, M // _NOB, n_box), jnp.float32),
            pltpu.VMEM((_NOB, M // _NOB, n_obj), jnp.float32),
            pltpu.SemaphoreType.DMA((_NC,)),
            pltpu.SemaphoreType.DMA((_NC, 3)),
        ],
        compiler_params=pltpu.CompilerParams(
            vmem_limit_bytes=67108864,
        ),
    )(x, W, b)
    return (cls_out, box_out, obj_out)


# speedup vs baseline: 2.6078x; 2.4243x over previous
import jax
import jax.numpy as jnp
from jax.experimental import pallas as pl
from jax.experimental.pallas import tpu as pltpu

_BM = 2048


def _heads_kernel(x_ref, wc_ref, wb_ref, wo_ref, b_ref,
                  cls_ref, box_ref, obj_ref):
    xb = x_ref[...]
    dims = (((1,), (1,)), ((), ()))
    cc = jax.lax.dot_general(wc_ref[...], xb, dims,
                             preferred_element_type=jnp.float32)
    cb = jax.lax.dot_general(wb_ref[...], xb, dims,
                             preferred_element_type=jnp.float32)
    co = jax.lax.dot_general(wo_ref[...], xb, dims,
                             preferred_element_type=jnp.float32)
    n_cls = cls_ref.shape[0]
    n_box = box_ref.shape[0]
    b = b_ref[...]
    cls_ref[...] = cc + b[:n_cls, :]
    box_ref[...] = cb + b[n_cls:n_cls + n_box, :]
    obj_ref[...] = co + b[n_cls + n_box:, :]


def kernel(x, W_cls, b_cls, W_box, b_box, W_obj, b_obj):
    M, K = x.shape
    n_cls = W_cls.shape[1]
    n_box = W_box.shape[1]
    n_obj = W_obj.shape[1]
    n_all = n_cls + n_box + n_obj

    WcT, WbT, WoT = W_cls.T, W_box.T, W_obj.T
    b_all = jnp.concatenate([b_cls, b_box, b_obj])[:, None]

    grid = (pl.cdiv(M, _BM),)
    cls_t, box_t, obj_t = pl.pallas_call(
        _heads_kernel,
        grid=grid,
        in_specs=[
            pl.BlockSpec((_BM, K), lambda i: (i, 0)),
            pl.BlockSpec((n_cls, K), lambda i: (0, 0)),
            pl.BlockSpec((n_box, K), lambda i: (0, 0)),
            pl.BlockSpec((n_obj, K), lambda i: (0, 0)),
            pl.BlockSpec((n_all, 1), lambda i: (0, 0)),
        ],
        out_specs=[
            pl.BlockSpec((n_cls, _BM), lambda i: (0, i)),
            pl.BlockSpec((n_box, _BM), lambda i: (0, i)),
            pl.BlockSpec((n_obj, _BM), lambda i: (0, i)),
        ],
        out_shape=[
            jax.ShapeDtypeStruct((n_cls, M), x.dtype),
            jax.ShapeDtypeStruct((n_box, M), x.dtype),
            jax.ShapeDtypeStruct((n_obj, M), x.dtype),
        ],
        compiler_params=pltpu.CompilerParams(
            dimension_semantics=("parallel",),
        ),
    )(x, WcT, WbT, WoT, b_all)
    return (cls_t.T, box_t.T, obj_t.T)


# speedup vs baseline: 3.0550x; 1.1715x over previous
import jax
import jax.numpy as jnp
from jax.experimental import pallas as pl
from jax.experimental.pallas import tpu as pltpu

_BM = 2048
_OFF_BOX = 24
_OFF_OBJ = 72
_NPAD = 80


def _heads_kernel(x_ref, wc_ref, wb_ref, wo_ref, b_ref,
                  cls_ref, box_ref, obj_ref, w_s):
    n_cls = cls_ref.shape[0]
    n_box = box_ref.shape[0]
    n_obj = obj_ref.shape[0]

    @pl.when(pl.program_id(0) == 0)
    def _init():
        w_s[...] = jnp.zeros_like(w_s)
        w_s[0:n_cls, :] = wc_ref[...]
        w_s[_OFF_BOX:_OFF_BOX + n_box, :] = wb_ref[...]
        w_s[_OFF_OBJ:_OFF_OBJ + n_obj, :] = wo_ref[...]

    acc = jax.lax.dot_general(
        w_s[...], x_ref[...], (((1,), (1,)), ((), ())),
        preferred_element_type=jnp.float32)
    acc = acc + jnp.transpose(b_ref[...])
    cls_ref[...] = acc[0:n_cls, :]
    box_ref[...] = acc[_OFF_BOX:_OFF_BOX + n_box, :]
    obj_ref[...] = acc[_OFF_OBJ:_OFF_OBJ + n_obj, :]


def kernel(x, W_cls, b_cls, W_box, b_box, W_obj, b_obj):
    M, K = x.shape
    n_cls = W_cls.shape[1]
    n_box = W_box.shape[1]
    n_obj = W_obj.shape[1]

    WcT, WbT, WoT = W_cls.T, W_box.T, W_obj.T
    zc = jnp.zeros((_OFF_BOX - n_cls,), dtype=x.dtype)
    zb = jnp.zeros((_OFF_OBJ - _OFF_BOX - n_box,), dtype=x.dtype)
    zo = jnp.zeros((_NPAD - _OFF_OBJ - n_obj,), dtype=x.dtype)
    b_pad = jnp.concatenate([b_cls, zc, b_box, zb, b_obj, zo])[None, :]

    grid = (pl.cdiv(M, _BM),)
    cls_t, box_t, obj_t = pl.pallas_call(
        _heads_kernel,
        grid=grid,
        in_specs=[
            pl.BlockSpec((_BM, K), lambda i: (i, 0)),
            pl.BlockSpec((n_cls, K), lambda i: (0, 0)),
            pl.BlockSpec((n_box, K), lambda i: (0, 0)),
            pl.BlockSpec((n_obj, K), lambda i: (0, 0)),
            pl.BlockSpec((1, _NPAD), lambda i: (0, 0)),
        ],
        out_specs=[
            pl.BlockSpec((n_cls, _BM), lambda i: (0, i)),
            pl.BlockSpec((n_box, _BM), lambda i: (0, i)),
            pl.BlockSpec((n_obj, _BM), lambda i: (0, i)),
        ],
        out_shape=[
            jax.ShapeDtypeStruct((n_cls, M), x.dtype),
            jax.ShapeDtypeStruct((n_box, M), x.dtype),
            jax.ShapeDtypeStruct((n_obj, M), x.dtype),
        ],
        scratch_shapes=[
            pltpu.VMEM((_NPAD, K), jnp.float32),
        ],
        compiler_params=pltpu.CompilerParams(
            dimension_semantics=("arbitrary",),
        ),
    )(x, WcT, WbT, WoT, b_pad)
    return (cls_t.T, box_t.T, obj_t.T)


# speedup vs baseline: 3.4806x; 1.1393x over previous
import jax
import jax.numpy as jnp
from jax.experimental import pallas as pl
from jax.experimental.pallas import tpu as pltpu

_BM = 4096
_OFF_BOX = 24
_OFF_OBJ = 72
_NPAD = 80


def _heads_kernel(x_ref, wc_ref, wb_ref, wo_ref, b_ref,
                  cls_ref, box_ref, obj_ref, w_s):
    n_cls = cls_ref.shape[0]
    n_box = box_ref.shape[0]
    n_obj = obj_ref.shape[0]

    @pl.when(pl.program_id(0) == 0)
    def _init():
        w_s[...] = jnp.zeros_like(w_s)
        w_s[0:n_cls, :] = wc_ref[...]
        w_s[_OFF_BOX:_OFF_BOX + n_box, :] = wb_ref[...]
        w_s[_OFF_OBJ:_OFF_OBJ + n_obj, :] = wo_ref[...]

    acc = jax.lax.dot_general(
        w_s[...], x_ref[...], (((1,), (1,)), ((), ())),
        preferred_element_type=jnp.float32)
    acc = acc + jnp.transpose(b_ref[...])
    cls_ref[...] = acc[0:n_cls, :]
    box_ref[...] = acc[_OFF_BOX:_OFF_BOX + n_box, :]
    obj_ref[...] = acc[_OFF_OBJ:_OFF_OBJ + n_obj, :]


def kernel(x, W_cls, b_cls, W_box, b_box, W_obj, b_obj):
    M, K = x.shape
    n_cls = W_cls.shape[1]
    n_box = W_box.shape[1]
    n_obj = W_obj.shape[1]

    WcT, WbT, WoT = W_cls.T, W_box.T, W_obj.T
    zc = jnp.zeros((_OFF_BOX - n_cls,), dtype=x.dtype)
    zb = jnp.zeros((_OFF_OBJ - _OFF_BOX - n_box,), dtype=x.dtype)
    zo = jnp.zeros((_NPAD - _OFF_OBJ - n_obj,), dtype=x.dtype)
    b_pad = jnp.concatenate([b_cls, zc, b_box, zb, b_obj, zo])[None, :]

    grid = (pl.cdiv(M, _BM),)
    cls_t, box_t, obj_t = pl.pallas_call(
        _heads_kernel,
        grid=grid,
        in_specs=[
            pl.BlockSpec((_BM, K), lambda i: (i, 0)),
            pl.BlockSpec((n_cls, K), lambda i: (0, 0)),
            pl.BlockSpec((n_box, K), lambda i: (0, 0)),
            pl.BlockSpec((n_obj, K), lambda i: (0, 0)),
            pl.BlockSpec((1, _NPAD), lambda i: (0, 0)),
        ],
        out_specs=[
            pl.BlockSpec((n_cls, _BM), lambda i: (0, i)),
            pl.BlockSpec((n_box, _BM), lambda i: (0, i)),
            pl.BlockSpec((n_obj, _BM), lambda i: (0, i)),
        ],
        out_shape=[
            jax.ShapeDtypeStruct((n_cls, M), x.dtype),
            jax.ShapeDtypeStruct((n_box, M), x.dtype),
            jax.ShapeDtypeStruct((n_obj, M), x.dtype),
        ],
        scratch_shapes=[
            pltpu.VMEM((_NPAD, K), jnp.float32),
        ],
        compiler_params=pltpu.CompilerParams(
            dimension_semantics=("arbitrary",),
        ),
    )(x, WcT, WbT, WoT, b_pad)
    return (cls_t.T, box_t.T, obj_t.T)


# speedup vs baseline: 3.7716x; 1.0836x over previous
import jax
import jax.numpy as jnp
from jax.experimental import pallas as pl
from jax.experimental.pallas import tpu as pltpu

_BM = 8192
_OFF_BOX = 24
_OFF_OBJ = 72
_NPAD = 80


def _heads_kernel(x_ref, wc_ref, wb_ref, wo_ref, b_ref,
                  cls_ref, box_ref, obj_ref, w_s):
    n_cls = cls_ref.shape[0]
    n_box = box_ref.shape[0]
    n_obj = obj_ref.shape[0]

    @pl.when(pl.program_id(0) == 0)
    def _init():
        w_s[...] = jnp.zeros_like(w_s)
        w_s[0:n_cls, :] = wc_ref[...]
        w_s[_OFF_BOX:_OFF_BOX + n_box, :] = wb_ref[...]
        w_s[_OFF_OBJ:_OFF_OBJ + n_obj, :] = wo_ref[...]

    acc = jax.lax.dot_general(
        w_s[...], x_ref[...], (((1,), (1,)), ((), ())),
        preferred_element_type=jnp.float32)
    acc = acc + jnp.transpose(b_ref[...])
    cls_ref[...] = acc[0:n_cls, :]
    box_ref[...] = acc[_OFF_BOX:_OFF_BOX + n_box, :]
    obj_ref[...] = acc[_OFF_OBJ:_OFF_OBJ + n_obj, :]


def kernel(x, W_cls, b_cls, W_box, b_box, W_obj, b_obj):
    M, K = x.shape
    n_cls = W_cls.shape[1]
    n_box = W_box.shape[1]
    n_obj = W_obj.shape[1]

    WcT, WbT, WoT = W_cls.T, W_box.T, W_obj.T
    zc = jnp.zeros((_OFF_BOX - n_cls,), dtype=x.dtype)
    zb = jnp.zeros((_OFF_OBJ - _OFF_BOX - n_box,), dtype=x.dtype)
    zo = jnp.zeros((_NPAD - _OFF_OBJ - n_obj,), dtype=x.dtype)
    b_pad = jnp.concatenate([b_cls, zc, b_box, zb, b_obj, zo])[None, :]

    grid = (pl.cdiv(M, _BM),)
    cls_t, box_t, obj_t = pl.pallas_call(
        _heads_kernel,
        grid=grid,
        in_specs=[
            pl.BlockSpec((_BM, K), lambda i: (i, 0)),
            pl.BlockSpec((n_cls, K), lambda i: (0, 0)),
            pl.BlockSpec((n_box, K), lambda i: (0, 0)),
            pl.BlockSpec((n_obj, K), lambda i: (0, 0)),
            pl.BlockSpec((1, _NPAD), lambda i: (0, 0)),
        ],
        out_specs=[
            pl.BlockSpec((n_cls, _BM), lambda i: (0, i)),
            pl.BlockSpec((n_box, _BM), lambda i: (0, i)),
            pl.BlockSpec((n_obj, _BM), lambda i: (0, i)),
        ],
        out_shape=[
            jax.ShapeDtypeStruct((n_cls, M), x.dtype),
            jax.ShapeDtypeStruct((n_box, M), x.dtype),
            jax.ShapeDtypeStruct((n_obj, M), x.dtype),
        ],
        scratch_shapes=[
            pltpu.VMEM((_NPAD, K), jnp.float32),
        ],
        compiler_params=pltpu.CompilerParams(
            dimension_semantics=("arbitrary",),
        ),
    )(x, WcT, WbT, WoT, b_pad)
    return (cls_t.T, box_t.T, obj_t.T)
